# baseline (speedup 1.0000x reference)
import functools

import jax
import jax.numpy as jnp
from jax import lax
from jax.experimental import pallas as pl
from jax.experimental.pallas import tpu as pltpu
from jax.experimental.pallas import tpu_sc as plsc

D = 128
NC = 2
NS = 16
NW = NC * NS
K = 128
F0 = 0.63
ACC_ROWS = 10240
STRIPE = ACC_ROWS // NS
PACK_BITS = 14


def _make_scatter(n0, n1):
  nmax = max(n0, n1)
  mesh = plsc.VectorSubcoreMesh(core_axis_name="c", subcore_axis_name="s",
                                num_cores=NC, num_subcores=NS)

  @functools.partial(
      pl.kernel,
      out_type=jax.ShapeDtypeStruct((NC, ACC_ROWS, D), jnp.float32),
      mesh=mesh,
      scratch_types=[
          pltpu.VMEM((nmax, K), jnp.int32),
          pltpu.VMEM((nmax, K), jnp.int32),
          pltpu.VMEM((2, K), jnp.int32),
          pltpu.VMEM((K, D), jnp.float32),
          pltpu.VMEM_SHARED((ACC_ROWS, D), jnp.float32),
          pltpu.SemaphoreType.DMA,
      ],
  )
  def scatter(pre_hbm, pk_hbm, ew_hbm, z_hbm, out_hbm,
              pk_v, ew_v, idx_v, rows_v, acc, sem):
    cid = lax.axis_index("c")
    sid = lax.axis_index("s")
    wid = sid * NC + cid
    pltpu.sync_copy(pk_hbm.at[wid], pk_v)
    pltpu.sync_copy(ew_hbm.at[wid], ew_v)
    pltpu.sync_copy(z_hbm, acc.at[pl.ds(sid * STRIPE, STRIPE)])
    plsc.subcore_barrier()

    n_mine = jnp.where(cid == 0, n0, n1)

    def chunk(c, carry):
      def unpack(g, cr):
        pk = pk_v[c, pl.ds(g * 16, 16)]
        idx_v[0, pl.ds(g * 16, 16)] = lax.bitwise_and(pk, (1 << PACK_BITS) - 1)
        idx_v[1, pl.ds(g * 16, 16)] = lax.shift_right_logical(pk, PACK_BITS)
        return cr

      lax.fori_loop(0, K // 16, unpack, 0)
      pltpu.async_copy(pre_hbm.at[idx_v.at[0]], rows_v, sem).wait()

      def group(g, carry2):
        ew16 = lax.bitcast_convert_type(ew_v[c, pl.ds(g * 16, 16)],
                                        jnp.float32)
        for j in range(16):
          w = ew16[j]
          e = g * 16 + j
          for f in range(D // 16):
            sl = pl.ds(f * 16, 16)
            rows_v[e, sl] = rows_v[e, sl] * w
        return carry2

      lax.fori_loop(0, K // 16, group, 0)
      pltpu.sync_copy(rows_v, acc.at[idx_v.at[1]], add=True)
      return carry

    lax.fori_loop(0, n_mine, chunk, 0)
    plsc.subcore_barrier()
    pltpu.sync_copy(acc.at[pl.ds(sid * STRIPE, STRIPE)],
                    out_hbm.at[cid, pl.ds(sid * STRIPE, STRIPE)])

  return scatter


def _mm_plain_body(x_ref, w_ref, o_ref):
  o_ref[...] = jnp.dot(x_ref[...], w_ref[...],
                       preferred_element_type=jnp.float32)


def _mm_fused_body(a_ref, b_ref, w_ref, o_ref):
  h = jnp.maximum(a_ref[...] + b_ref[...], 0.0)
  o_ref[...] = jnp.dot(h, w_ref[...], preferred_element_type=jnp.float32)


def _mm_fused_bias_body(a_ref, b_ref, w_ref, bias_ref, o_ref):
  h = jnp.maximum(a_ref[...] + b_ref[...], 0.0)
  o_ref[...] = (jnp.dot(h, w_ref[...], preferred_element_type=jnp.float32)
                + bias_ref[...])


_BM = 2000


def _matmul(x, w):
  m, k = x.shape
  n = w.shape[1]
  return pl.pallas_call(
      _mm_plain_body,
      grid=(m // _BM,),
      in_specs=[pl.BlockSpec((_BM, k), lambda i: (i, 0)),
                pl.BlockSpec((k, n), lambda i: (0, 0))],
      out_specs=pl.BlockSpec((_BM, n), lambda i: (i, 0)),
      out_shape=jax.ShapeDtypeStruct((m, n), jnp.float32),
  )(x, w)


def _fused_matmul(a, b, w):
  m, k = a.shape
  n = w.shape[1]
  return pl.pallas_call(
      _mm_fused_body,
      grid=(m // _BM,),
      in_specs=[pl.BlockSpec((_BM, k), lambda i: (i, 0)),
                pl.BlockSpec((_BM, k), lambda i: (i, 0)),
                pl.BlockSpec((k, n), lambda i: (0, 0))],
      out_specs=pl.BlockSpec((_BM, n), lambda i: (i, 0)),
      out_shape=jax.ShapeDtypeStruct((m, n), jnp.float32),
  )(a, b, w)


def _fused_matmul_bias(a, b, w, bias):
  m, k = a.shape
  n = w.shape[1]
  return pl.pallas_call(
      _mm_fused_bias_body,
      grid=(m // _BM,),
      in_specs=[pl.BlockSpec((_BM, k), lambda i: (i, 0)),
                pl.BlockSpec((_BM, k), lambda i: (i, 0)),
                pl.BlockSpec((k, n), lambda i: (0, 0)),
                pl.BlockSpec((1, n), lambda i: (0, 0))],
      out_specs=pl.BlockSpec((_BM, n), lambda i: (i, 0)),
      out_shape=jax.ShapeDtypeStruct((m, n), jnp.float32),
  )(a, b, w, bias)


def _slab(v, n0, n1, nmax):
  a = v[:NS * n0 * K].reshape(NS, n0 * K)
  b = v[NS * n0 * K:].reshape(NS, n1 * K)
  width = nmax * K
  a = jnp.pad(a, ((0, 0), (0, width - a.shape[1])))
  b = jnp.pad(b, ((0, 0), (0, width - b.shape[1])))
  return jnp.stack([a, b], axis=1).reshape(NW, nmax, K)


def kernel(x, edge_index, edge_weight, W0, W1, Wp, bp):
  n_edges = edge_index.shape[1]
  npair = -(-n_edges // (NS * K))
  n0 = max(1, min(npair - 1, round(npair * F0)))
  n1 = npair - n0
  nmax = max(n0, n1)
  pad = NS * npair * K - n_edges

  src = jnp.pad(edge_index[0].astype(jnp.int32), (0, pad))
  dst = jnp.pad(edge_index[1].astype(jnp.int32), (0, pad))
  ew = lax.bitcast_convert_type(
      jnp.pad(edge_weight.astype(jnp.float32), (0, pad)), jnp.int32)
  packed = jnp.bitwise_or(src, jnp.left_shift(dst, PACK_BITS))
  pk = _slab(packed, n0, n1, nmax)
  ews = _slab(ew, n0, n1, nmax)
  zeros = jnp.zeros((STRIPE, D), jnp.float32)

  scatter = _make_scatter(n0, n1)

  n = x.shape[0]
  pre0 = _matmul(x, W0)
  p = scatter(pre0, pk, ews, zeros)
  pre1 = _fused_matmul(p[0, :n], p[1, :n], W1)
  q = scatter(pre1, pk, ews, zeros)

  out_dim = Wp.shape[1]
  wp = jnp.pad(Wp, ((0, 0), (0, D - out_dim)))
  bpad = jnp.pad(bp, (0, D - out_dim)).reshape(1, D)
  out = _fused_matmul_bias(q[0, :n], q[1, :n], wp, bpad)
  return out[:, :out_dim]


# speedup vs baseline: 1.0520x; 1.0520x over previous
import functools

import jax
import jax.numpy as jnp
from jax import lax
from jax.experimental import pallas as pl
from jax.experimental.pallas import tpu as pltpu
from jax.experimental.pallas import tpu_sc as plsc

D = 128
NC = 2
NS = 16
NW = NC * NS
K = 128
F0 = 0.58
ACC_ROWS = 10240
STRIPE = ACC_ROWS // NS
PACK_BITS = 14


def _make_scatter(n0, n1):
  nmax = max(n0, n1)
  mesh = plsc.VectorSubcoreMesh(core_axis_name="c", subcore_axis_name="s",
                                num_cores=NC, num_subcores=NS)

  @functools.partial(
      pl.kernel,
      out_type=jax.ShapeDtypeStruct((NC, ACC_ROWS, D), jnp.float32),
      mesh=mesh,
      scratch_types=[
          pltpu.VMEM((nmax, K), jnp.int32),
          pltpu.VMEM((nmax, K), jnp.int32),
          pltpu.VMEM((2, K), jnp.int32),
          pltpu.VMEM((K, D), jnp.float32),
          pltpu.VMEM_SHARED((ACC_ROWS, D), jnp.float32),
          pltpu.SemaphoreType.DMA,
      ],
  )
  def scatter(pre_hbm, pk_hbm, ew_hbm, z_hbm, out_hbm,
              pk_v, ew_v, idx_v, rows_v, acc, sem):
    cid = lax.axis_index("c")
    sid = lax.axis_index("s")
    wid = sid * NC + cid
    pltpu.sync_copy(pk_hbm.at[wid], pk_v)
    pltpu.sync_copy(ew_hbm.at[wid], ew_v)
    pltpu.sync_copy(z_hbm, acc.at[pl.ds(sid * STRIPE, STRIPE)])
    plsc.subcore_barrier()

    n_mine = jnp.where(cid == 0, n0, n1)

    def chunk(c, carry):
      def unpack(g, cr):
        pk = pk_v[c, pl.ds(g * 16, 16)]
        idx_v[0, pl.ds(g * 16, 16)] = lax.bitwise_and(pk, (1 << PACK_BITS) - 1)
        idx_v[1, pl.ds(g * 16, 16)] = lax.shift_right_logical(pk, PACK_BITS)
        return cr

      lax.fori_loop(0, K // 16, unpack, 0)
      pltpu.async_copy(pre_hbm.at[idx_v.at[0]], rows_v, sem).wait()

      def group(g, carry2):
        ew16 = lax.bitcast_convert_type(ew_v[c, pl.ds(g * 16, 16)],
                                        jnp.float32)
        for j in range(16):
          w = ew16[j]
          e = g * 16 + j
          for f in range(D // 16):
            sl = pl.ds(f * 16, 16)
            rows_v[e, sl] = rows_v[e, sl] * w
        return carry2

      lax.fori_loop(0, K // 16, group, 0)
      pltpu.sync_copy(rows_v, acc.at[idx_v.at[1]], add=True)
      return carry

    lax.fori_loop(0, n_mine, chunk, 0)
    plsc.subcore_barrier()
    pltpu.sync_copy(acc.at[pl.ds(sid * STRIPE, STRIPE)],
                    out_hbm.at[cid, pl.ds(sid * STRIPE, STRIPE)])

  return scatter


def _mm_plain_body(x_ref, w_ref, o_ref):
  o_ref[...] = jnp.dot(x_ref[...], w_ref[...],
                       preferred_element_type=jnp.float32)


def _mm_fused_body(a_ref, b_ref, w_ref, o_ref):
  h = jnp.maximum(a_ref[...] + b_ref[...], 0.0)
  o_ref[...] = jnp.dot(h, w_ref[...], preferred_element_type=jnp.float32)


def _mm_fused_bias_body(a_ref, b_ref, w_ref, bias_ref, o_ref):
  h = jnp.maximum(a_ref[...] + b_ref[...], 0.0)
  o_ref[...] = (jnp.dot(h, w_ref[...], preferred_element_type=jnp.float32)
                + bias_ref[...])


_BM = 2000


def _matmul(x, w):
  m, k = x.shape
  n = w.shape[1]
  return pl.pallas_call(
      _mm_plain_body,
      grid=(m // _BM,),
      in_specs=[pl.BlockSpec((_BM, k), lambda i: (i, 0)),
                pl.BlockSpec((k, n), lambda i: (0, 0))],
      out_specs=pl.BlockSpec((_BM, n), lambda i: (i, 0)),
      out_shape=jax.ShapeDtypeStruct((m, n), jnp.float32),
  )(x, w)


def _fused_matmul(a, b, w):
  m, k = a.shape
  n = w.shape[1]
  return pl.pallas_call(
      _mm_fused_body,
      grid=(m // _BM,),
      in_specs=[pl.BlockSpec((_BM, k), lambda i: (i, 0)),
                pl.BlockSpec((_BM, k), lambda i: (i, 0)),
                pl.BlockSpec((k, n), lambda i: (0, 0))],
      out_specs=pl.BlockSpec((_BM, n), lambda i: (i, 0)),
      out_shape=jax.ShapeDtypeStruct((m, n), jnp.float32),
  )(a, b, w)


def _fused_matmul_bias(a, b, w, bias):
  m, k = a.shape
  n = w.shape[1]
  return pl.pallas_call(
      _mm_fused_bias_body,
      grid=(m // _BM,),
      in_specs=[pl.BlockSpec((_BM, k), lambda i: (i, 0)),
                pl.BlockSpec((_BM, k), lambda i: (i, 0)),
                pl.BlockSpec((k, n), lambda i: (0, 0)),
                pl.BlockSpec((1, n), lambda i: (0, 0))],
      out_specs=pl.BlockSpec((_BM, n), lambda i: (i, 0)),
      out_shape=jax.ShapeDtypeStruct((m, n), jnp.float32),
  )(a, b, w, bias)


def _slab(v, n0, n1, nmax):
  a = v[:NS * n0 * K].reshape(NS, n0 * K)
  b = v[NS * n0 * K:].reshape(NS, n1 * K)
  width = nmax * K
  a = jnp.pad(a, ((0, 0), (0, width - a.shape[1])))
  b = jnp.pad(b, ((0, 0), (0, width - b.shape[1])))
  return jnp.stack([a, b], axis=1).reshape(NW, nmax, K)


def kernel(x, edge_index, edge_weight, W0, W1, Wp, bp):
  n_edges = edge_index.shape[1]
  npair = -(-n_edges // (NS * K))
  n0 = max(1, min(npair - 1, round(npair * F0)))
  n1 = npair - n0
  nmax = max(n0, n1)
  pad = NS * npair * K - n_edges

  src = jnp.pad(edge_index[0].astype(jnp.int32), (0, pad))
  dst = jnp.pad(edge_index[1].astype(jnp.int32), (0, pad))
  ew = lax.bitcast_convert_type(
      jnp.pad(edge_weight.astype(jnp.float32), (0, pad)), jnp.int32)
  packed = jnp.bitwise_or(src, jnp.left_shift(dst, PACK_BITS))
  pk = _slab(packed, n0, n1, nmax)
  ews = _slab(ew, n0, n1, nmax)
  zeros = jnp.zeros((STRIPE, D), jnp.float32)

  scatter = _make_scatter(n0, n1)

  n = x.shape[0]
  pre0 = _matmul(x, W0)
  p = scatter(pre0, pk, ews, zeros)
  pre1 = _fused_matmul(p[0, :n], p[1, :n], W1)
  q = scatter(pre1, pk, ews, zeros)

  out_dim = Wp.shape[1]
  wp = jnp.pad(Wp, ((0, 0), (0, D - out_dim)))
  bpad = jnp.pad(bp, (0, D - out_dim)).reshape(1, D)
  out = _fused_matmul_bias(q[0, :n], q[1, :n], wp, bpad)
  return out[:, :out_dim]
